# baseline (speedup 1.0000x reference)
import functools

import jax
import jax.numpy as jnp
from jax import lax
from jax.experimental import pallas as pl
from jax.experimental.pallas import tpu as pltpu
from jax.experimental.pallas import tpu_sc as plsc

D = 128
B = 4096
NC = 2
NS = 16
NW = NC * NS
BPW = B // NW
L = 16
CHUNKS = D // L


def _sc_body(table_hbm, uid_hbm, item_hbm, w_hbm, out_hbm,
             idx_v, rows_v, item_v, w_v, out_v, sem):
    wid = lax.axis_index("s") * NC + lax.axis_index("c")
    base = wid * BPW

    pltpu.sync_copy(uid_hbm.at[pl.ds(base, BPW)], idx_v)
    gather = pltpu.async_copy(table_hbm.at[idx_v], rows_v, sem)
    pltpu.sync_copy(item_hbm.at[pl.ds(base, BPW)], item_v)
    pltpu.sync_copy(w_hbm, w_v)
    gather.wait()

    def row(r, carry):
        acc = rows_v[r, pl.ds(0, L)] * w_v[pl.ds(0, L)]
        for c in range(1, CHUNKS):
            acc = acc + rows_v[r, pl.ds(c * L, L)] * w_v[pl.ds(c * L, L)]
        for c in range(CHUNKS):
            acc = acc + item_v[r, pl.ds(c * L, L)] * w_v[pl.ds(D + c * L, L)]
        out_v[r, pl.ds(0, L)] = acc
        return carry

    lax.fori_loop(0, BPW, row, 0)
    pltpu.sync_copy(out_v, out_hbm.at[pl.ds(base, BPW)])


@functools.cache
def _sc_partials():
    return pl.kernel(
        _sc_body,
        out_type=jax.ShapeDtypeStruct((B, L), jnp.float32),
        mesh=plsc.VectorSubcoreMesh(core_axis_name="c", subcore_axis_name="s"),
        scratch_types=[
            pltpu.VMEM((BPW,), jnp.int32),
            pltpu.VMEM((BPW, D), jnp.float32),
            pltpu.VMEM((BPW, D), jnp.float32),
            pltpu.VMEM((2 * D,), jnp.float32),
            pltpu.VMEM((BPW, L), jnp.float32),
            pltpu.SemaphoreType.DMA,
        ],
    )


def _tc_body(part_ref, ones_ref, b_ref, out_ref):
    z = jnp.dot(part_ref[...], ones_ref[...],
                preferred_element_type=jnp.float32)
    out_ref[...] = 1.0 / (1.0 + jnp.exp(-(z + b_ref[...])))


def _tc_finish(partials, b11):
    ones = jnp.ones((L, 1), jnp.float32)
    return pl.pallas_call(
        _tc_body,
        out_shape=jax.ShapeDtypeStruct((B, 1), jnp.float32),
    )(partials, ones, b11)


def kernel(user_id, item_emb, user_table, W, b):
    uid = user_id.astype(jnp.int32)
    partials = _sc_partials()(user_table, uid, item_emb, W.reshape(2 * D))
    return _tc_finish(partials, b.reshape(1, 1))


# speedup vs baseline: 1.0448x; 1.0448x over previous
import functools

import jax
import jax.numpy as jnp
from jax import lax
from jax.experimental import pallas as pl
from jax.experimental.pallas import tpu as pltpu
from jax.experimental.pallas import tpu_sc as plsc

D = 128
B = 4096
NC = 2
NS = 16
NW = NC * NS
BPW = B // NW
L = 16
CHUNKS = D // L


def _sc_body(table_hbm, uid_hbm, item_hbm, w_hbm, out_hbm,
             idx_v, rows_v, item_v, w_v, out_v, sem):
    wid = lax.axis_index("s") * NC + lax.axis_index("c")
    base = wid * BPW

    pltpu.sync_copy(uid_hbm.at[pl.ds(base, BPW)], idx_v)
    gather = pltpu.async_copy(table_hbm.at[idx_v], rows_v, sem)
    pltpu.sync_copy(item_hbm.at[pl.ds(base, BPW)], item_v)
    pltpu.sync_copy(w_hbm, w_v)
    gather.wait()

    wu = [w_v[pl.ds(c * L, L)] for c in range(CHUNKS)]
    wi = [w_v[pl.ds(D + c * L, L)] for c in range(CHUNKS)]

    def row(r):
        acc = rows_v[r, pl.ds(0, L)] * wu[0]
        for c in range(1, CHUNKS):
            acc = acc + rows_v[r, pl.ds(c * L, L)] * wu[c]
        for c in range(CHUNKS):
            acc = acc + item_v[r, pl.ds(c * L, L)] * wi[c]
        out_v[r, pl.ds(0, L)] = acc

    plsc.parallel_loop(0, BPW, 1, unroll=8)(row)
    pltpu.sync_copy(out_v, out_hbm.at[pl.ds(base, BPW)])


@functools.cache
def _sc_partials():
    return pl.kernel(
        _sc_body,
        out_type=jax.ShapeDtypeStruct((B, L), jnp.float32),
        mesh=plsc.VectorSubcoreMesh(core_axis_name="c", subcore_axis_name="s"),
        scratch_types=[
            pltpu.VMEM((BPW,), jnp.int32),
            pltpu.VMEM((BPW, D), jnp.float32),
            pltpu.VMEM((BPW, D), jnp.float32),
            pltpu.VMEM((2 * D,), jnp.float32),
            pltpu.VMEM((BPW, L), jnp.float32),
            pltpu.SemaphoreType.DMA,
        ],
    )


def _tc_body(part_ref, ones_ref, b_ref, out_ref):
    z = jnp.dot(part_ref[...], ones_ref[...],
                preferred_element_type=jnp.float32)
    out_ref[...] = 1.0 / (1.0 + jnp.exp(-(z + b_ref[...])))


def _tc_finish(partials, b11):
    ones = jnp.ones((L, 1), jnp.float32)
    return pl.pallas_call(
        _tc_body,
        out_shape=jax.ShapeDtypeStruct((B, 1), jnp.float32),
    )(partials, ones, b11)


def kernel(user_id, item_emb, user_table, W, b):
    uid = user_id.astype(jnp.int32)
    partials = _sc_partials()(user_table, uid, item_emb, W.reshape(2 * D))
    return _tc_finish(partials, b.reshape(1, 1))
